# Initial kernel scaffold; baseline (speedup 1.0000x reference)
import jax
import jax.numpy as jnp
from jax.experimental import pallas as pl


def kernel(sparse_idx, dense_feats, deep_tables, wide_tables, W1, b1, W2, b2, W3, b3, W4, b4, Ww, bw):
    raise NotImplementedError("write your pallas kernel here")


# speedup vs baseline: 1.9826x; 1.9826x over previous
import functools

import jax
import jax.numpy as jnp
from jax import lax
from jax.experimental import pallas as pl
from jax.experimental.pallas import tpu as pltpu
from jax.experimental.pallas import tpu_sc as plsc

B = 4096
F_SP = 26
F_DN = 13
V = 100000
D = 32
N = B * F_SP

NW = 32
ROWS_PER_W = N // NW


def _sc_gather_body(deep_hbm, wide_hbm, gidx_hbm, deep_out, wide_out,
                    idx_v, rows_v, wrow_v, sem):
    wid = lax.axis_index("s") * 2 + lax.axis_index("c")
    base = wid * ROWS_PER_W
    pltpu.sync_copy(gidx_hbm.at[pl.ds(base, ROWS_PER_W)], idx_v)
    cp_d = pltpu.async_copy(deep_hbm.at[idx_v], rows_v, sem)
    cp_w = pltpu.async_copy(wide_hbm.at[idx_v], wrow_v, sem)
    cp_d.wait()
    cp_w.wait()
    pltpu.sync_copy(rows_v, deep_out.at[pl.ds(base, ROWS_PER_W)])
    pltpu.sync_copy(wrow_v, wide_out.at[pl.ds(base, ROWS_PER_W)])


def _make_sc_gather():
    mesh = plsc.VectorSubcoreMesh(core_axis_name="c", subcore_axis_name="s")
    return pl.kernel(
        _sc_gather_body,
        mesh=mesh,
        out_type=[
            jax.ShapeDtypeStruct((N, D), jnp.float32),
            jax.ShapeDtypeStruct((N,), jnp.float32),
        ],
        scratch_types=[
            pltpu.VMEM((ROWS_PER_W,), jnp.int32),
            pltpu.VMEM((ROWS_PER_W, D), jnp.float32),
            pltpu.VMEM((ROWS_PER_W,), jnp.float32),
            pltpu.SemaphoreType.DMA,
        ],
        compiler_params=pltpu.CompilerParams(use_tc_tiling_on_sc=False),
    )


BT = 1024


def _mlp_body(demb, dense, wemb, w1a, w1b, b1, w2, b2, w3, b3, w4, b4,
              wwa, wwb, bw, h1o, h2o, h3o, dlo, wlo, flo):
    x = dense[...]
    acc = jnp.dot(demb[...], w1a[...], preferred_element_type=jnp.float32)
    acc += jnp.dot(x, w1b[...], preferred_element_type=jnp.float32)
    h1 = jnp.maximum(acc + b1[...], 0.0)
    h1o[...] = h1
    h2 = jnp.maximum(
        jnp.dot(h1, w2[...], preferred_element_type=jnp.float32) + b2[...], 0.0)
    h2o[...] = h2
    h3 = jnp.maximum(
        jnp.dot(h2, w3[...], preferred_element_type=jnp.float32) + b3[...], 0.0)
    h3o[...] = h3
    dl = jnp.dot(h3, w4[...], preferred_element_type=jnp.float32) + b4[...]
    dlo[...] = dl
    wl = (jnp.dot(wemb[...], wwa[...], preferred_element_type=jnp.float32)
          + jnp.dot(x, wwb[...], preferred_element_type=jnp.float32) + bw[...])
    wlo[...] = wl
    flo[...] = dl + wl


def _make_mlp():
    grid = (B // BT,)

    def tile(i):
        return (i, 0)

    def fixed(i):
        return (0, 0)

    in_specs = [
        pl.BlockSpec((BT, F_SP * D), tile),
        pl.BlockSpec((BT, F_DN), tile),
        pl.BlockSpec((BT, F_SP), tile),
        pl.BlockSpec((F_SP * D, 1024), fixed),
        pl.BlockSpec((F_DN, 1024), fixed),
        pl.BlockSpec((1, 1024), fixed),
        pl.BlockSpec((1024, 512), fixed),
        pl.BlockSpec((1, 512), fixed),
        pl.BlockSpec((512, 256), fixed),
        pl.BlockSpec((1, 256), fixed),
        pl.BlockSpec((256, 1), fixed),
        pl.BlockSpec((1, 1), fixed),
        pl.BlockSpec((F_SP, 1), fixed),
        pl.BlockSpec((F_DN, 1), fixed),
        pl.BlockSpec((1, 1), fixed),
    ]
    out_specs = [
        pl.BlockSpec((BT, 1024), tile),
        pl.BlockSpec((BT, 512), tile),
        pl.BlockSpec((BT, 256), tile),
        pl.BlockSpec((BT, 1), tile),
        pl.BlockSpec((BT, 1), tile),
        pl.BlockSpec((BT, 1), tile),
    ]
    out_shape = [
        jax.ShapeDtypeStruct((B, 1024), jnp.float32),
        jax.ShapeDtypeStruct((B, 512), jnp.float32),
        jax.ShapeDtypeStruct((B, 256), jnp.float32),
        jax.ShapeDtypeStruct((B, 1), jnp.float32),
        jax.ShapeDtypeStruct((B, 1), jnp.float32),
        jax.ShapeDtypeStruct((B, 1), jnp.float32),
    ]
    return pl.pallas_call(
        _mlp_body,
        grid=grid,
        in_specs=in_specs,
        out_specs=out_specs,
        out_shape=out_shape,
    )


def kernel(sparse_idx, dense_feats, deep_tables, wide_tables,
           W1, b1, W2, b2, W3, b3, W4, b4, Ww, bw):
    gidx = (sparse_idx
            + (jnp.arange(F_SP, dtype=jnp.int32) * V)[None, :]).reshape(N)
    deep_flat = deep_tables.reshape(F_SP * V, D)
    wide_flat = wide_tables.reshape(F_SP * V)

    deep_g, wide_g = _make_sc_gather()(deep_flat, wide_flat, gidx)
    deep_emb = deep_g.reshape(B, F_SP * D)
    wide_emb = wide_g.reshape(B, F_SP)

    h1, h2, h3, dl, wl, fl = _make_mlp()(
        deep_emb, dense_feats, wide_emb,
        W1[:F_SP * D], W1[F_SP * D:], b1.reshape(1, -1),
        W2, b2.reshape(1, -1), W3, b3.reshape(1, -1),
        W4, b4.reshape(1, -1), Ww[:F_SP], Ww[F_SP:], bw.reshape(1, -1))
    return (h1, h2, h3, dl, wl, fl)
